# Initial kernel scaffold; baseline (speedup 1.0000x reference)
import jax
import jax.numpy as jnp
from jax.experimental import pallas as pl


def kernel(edge_lists, node_states, W, b):
    raise NotImplementedError("write your pallas kernel here")


# speedup vs baseline: 1.4420x; 1.4420x over previous
import functools

import jax
import jax.numpy as jnp
from jax import lax
from jax.experimental import pallas as pl
from jax.experimental.pallas import tpu as pltpu
from jax.experimental.pallas import tpu_sc as plsc

_NC = 2
_NS = 16
_EB = 128
_CG = 8


def _transform_table(node_states, W, b, E):
    N, D = node_states.shape
    RB = 2000
    assert N % RB == 0

    def body(x_ref, w_ref, b_ref, o_ref):
        acc = lax.dot_general(
            x_ref[...], w_ref[...], (((1,), (1,)), ((), ())),
            preferred_element_type=jnp.float32)
        o_ref[...] = (acc + b_ref[0])[None]

    out = pl.pallas_call(
        body,
        grid=(E, N // RB),
        in_specs=[
            pl.BlockSpec((RB, D), lambda e, r: (r, 0)),
            pl.BlockSpec((D, D), lambda e, r: (e, 0)),
            pl.BlockSpec((1, 1, D), lambda e, r: (e, 0, 0)),
        ],
        out_specs=pl.BlockSpec((1, RB, D), lambda e, r: (e, r, 0)),
        out_shape=jax.ShapeDtypeStruct((E, N, D), jnp.float32),
    )(node_states, W, b.reshape(E, 1, D))
    return out.reshape(E * N, D)


def _scatter_accumulate(src, tgt, tgt_hi, tgt_lo, table, n_pad, D):
    NC, NS, NB, EB = src.shape
    rpt = n_pad // NS
    nck = rpt // EB
    crows = n_pad // EB
    crows_pad = -(-crows // 8) * 8
    mesh = plsc.VectorSubcoreMesh(core_axis_name="c", subcore_axis_name="s")

    @functools.partial(
        pl.kernel,
        mesh=mesh,
        out_type=[
            jax.ShapeDtypeStruct((NC, n_pad, D), jnp.float32),
            jax.ShapeDtypeStruct((NC, crows, EB), jnp.float32),
        ],
        scratch_types=[
            pltpu.VMEM_SHARED((n_pad, D), jnp.float32),
            pltpu.VMEM_SHARED((crows_pad, EB), jnp.float32),
            pltpu.VMEM((_CG, EB), jnp.int32),
            pltpu.VMEM((_CG, EB), jnp.int32),
            pltpu.VMEM((_CG, EB), jnp.int32),
            pltpu.VMEM((_CG, EB), jnp.int32),
            pltpu.VMEM((EB, D), jnp.float32),
            pltpu.VMEM((EB, EB), jnp.float32),
        ],
    )
    def sc_kernel(src_hbm, tgt_hbm, hi_hbm, lo_hbm, table_hbm, zero_hbm,
                  id_hbm, acc_out, cnt_out,
                  acc_sh, cnt_sh, src_v, tgt_v, hi_v, lo_v, rows_v, oh_v):
        c = lax.axis_index("c")
        s = lax.axis_index("s")
        t0 = s * rpt
        pltpu.sync_copy(zero_hbm, rows_v)
        for k in range(nck):
            pltpu.sync_copy(rows_v.at[pl.ds(0, EB)],
                            acc_sh.at[pl.ds(t0 + k * EB, EB)])

        @pl.when(s < crows_pad // 8)
        def _zero_cnt_sh():
            pltpu.sync_copy(rows_v.at[pl.ds(0, 8), pl.ds(0, EB)],
                            cnt_sh.at[pl.ds(s * 8, 8)])

        plsc.subcore_barrier()

        def chunk_body(g, carry):
            pltpu.sync_copy(src_hbm.at[c, s, pl.ds(g * _CG, _CG)], src_v)
            pltpu.sync_copy(tgt_hbm.at[c, s, pl.ds(g * _CG, _CG)], tgt_v)
            pltpu.sync_copy(hi_hbm.at[c, s, pl.ds(g * _CG, _CG)], hi_v)
            pltpu.sync_copy(lo_hbm.at[c, s, pl.ds(g * _CG, _CG)], lo_v)
            for j in range(_CG):
                pltpu.sync_copy(table_hbm.at[src_v.at[j]], rows_v)
                pltpu.sync_copy(rows_v, acc_sh.at[tgt_v.at[j]], add=True)
                pltpu.sync_copy(id_hbm.at[lo_v.at[j]], oh_v)
                pltpu.sync_copy(oh_v, cnt_sh.at[hi_v.at[j]], add=True)
            return carry

        lax.fori_loop(0, NB // _CG, chunk_body, 0)
        plsc.subcore_barrier()
        for k in range(nck):
            pltpu.sync_copy(acc_sh.at[pl.ds(t0 + k * EB, EB)],
                            rows_v.at[pl.ds(0, EB)])
            pltpu.sync_copy(rows_v.at[pl.ds(0, EB)],
                            acc_out.at[c, pl.ds(t0 + k * EB, EB)])

        @pl.when(s < crows // 8)
        def _drain_cnt():
            pltpu.sync_copy(cnt_sh.at[pl.ds(s * 8, 8)],
                            oh_v.at[pl.ds(0, 8)])
            pltpu.sync_copy(oh_v.at[pl.ds(0, 8)],
                            cnt_out.at[c, pl.ds(s * 8, 8)])

    zero = jnp.zeros((_EB, D), jnp.float32)
    id128 = jnp.eye(_EB, dtype=jnp.float32)
    return sc_kernel(src, tgt, tgt_hi, tgt_lo, table, zero, id128)


def _normalize(acc, cnt, n_pad, D):
    RB = 1024
    assert n_pad % RB == 0

    def body(a_ref, c_ref, o_ref):
        p = a_ref[0] + a_ref[1]
        n = c_ref[0] + c_ref[1]
        div = jnp.where(n == 0.0, 1.0, n)
        o_ref[...] = p / div + 1e-8

    return pl.pallas_call(
        body,
        grid=(n_pad // RB,),
        in_specs=[
            pl.BlockSpec((2, RB, D), lambda r: (0, r, 0)),
            pl.BlockSpec((2, RB, 1), lambda r: (0, r, 0)),
        ],
        out_specs=pl.BlockSpec((RB, D), lambda r: (r, 0)),
        out_shape=jax.ShapeDtypeStruct((n_pad, D), jnp.float32),
    )(acc, cnt)


def kernel(edge_lists, node_states, W, b):
    E, M, _ = edge_lists.shape
    N, D = node_states.shape

    table = _transform_table(node_states, W, b, E)

    el = edge_lists.astype(jnp.int32)
    src = (el[:, :, 0] + (jnp.arange(E, dtype=jnp.int32) * N)[:, None]).reshape(-1)
    tgt = el[:, :, 1].reshape(-1)

    lane_total = _NC * _NS * _EB * _CG
    m_tot = E * M
    m_pad = -(-m_tot // lane_total) * lane_total
    n_pad = -(-(N + 1) // (_NS * _EB)) * (_NS * _EB)
    src = jnp.pad(src, (0, m_pad - m_tot))
    tgt = jnp.pad(tgt, (0, m_pad - m_tot), constant_values=N)
    NB = m_pad // (_NC * _NS * _EB)
    shape4 = (_NC, _NS, NB, _EB)
    src = src.reshape(shape4)
    tgt = tgt.reshape(shape4)
    tgt_hi = jax.lax.shift_right_logical(tgt, 7)
    tgt_lo = jax.lax.bitwise_and(tgt, 127)

    acc, cnt = _scatter_accumulate(src, tgt, tgt_hi, tgt_lo, table, n_pad, D)
    out = _normalize(acc, cnt.reshape(_NC, n_pad, 1), n_pad, D)
    return out[:N]


# speedup vs baseline: 1.5578x; 1.0803x over previous
import functools

import jax
import jax.numpy as jnp
from jax import lax
from jax.experimental import pallas as pl
from jax.experimental.pallas import tpu as pltpu
from jax.experimental.pallas import tpu_sc as plsc

_NC = 2
_NS = 16
_EB = 128
_CG = 8


def _transform_table(node_states, W, b, E):
    N, D = node_states.shape
    RB = 2000
    assert N % RB == 0

    def body(x_ref, w_ref, b_ref, o_ref):
        acc = lax.dot_general(
            x_ref[...], w_ref[...], (((1,), (1,)), ((), ())),
            preferred_element_type=jnp.float32)
        o_ref[...] = (acc + b_ref[0])[None]

    out = pl.pallas_call(
        body,
        grid=(E, N // RB),
        in_specs=[
            pl.BlockSpec((RB, D), lambda e, r: (r, 0)),
            pl.BlockSpec((D, D), lambda e, r: (e, 0)),
            pl.BlockSpec((1, 1, D), lambda e, r: (e, 0, 0)),
        ],
        out_specs=pl.BlockSpec((1, RB, D), lambda e, r: (e, r, 0)),
        out_shape=jax.ShapeDtypeStruct((E, N, D), jnp.float32),
    )(node_states, W, b.reshape(E, 1, D))
    return out.reshape(E * N, D)


def _scatter_accumulate(src, tgt, tgt_hi, tgt_lo, table, n_pad, D):
    NC, NS, NB, EB = src.shape
    rpt = n_pad // NS
    nck = rpt // EB
    crows = n_pad // EB
    crows_pad = -(-crows // 8) * 8
    mesh = plsc.VectorSubcoreMesh(core_axis_name="c", subcore_axis_name="s")

    @functools.partial(
        pl.kernel,
        mesh=mesh,
        out_type=[
            jax.ShapeDtypeStruct((NC, n_pad, D), jnp.float32),
            jax.ShapeDtypeStruct((NC, crows, EB), jnp.float32),
        ],
        scratch_types=[
            pltpu.VMEM_SHARED((n_pad, D), jnp.float32),
            pltpu.VMEM_SHARED((crows_pad, EB), jnp.float32),
            pltpu.VMEM((_CG, EB), jnp.int32),
            pltpu.VMEM((_CG, EB), jnp.int32),
            pltpu.VMEM((_CG, EB), jnp.int32),
            pltpu.VMEM((_CG, EB), jnp.int32),
            pltpu.VMEM((EB, D), jnp.float32),
            pltpu.VMEM((EB, EB), jnp.float32),
            pltpu.SemaphoreType.DMA,
            pltpu.SemaphoreType.DMA,
            pltpu.SemaphoreType.DMA,
            pltpu.SemaphoreType.DMA,
        ],
    )
    def sc_kernel(src_hbm, tgt_hbm, hi_hbm, lo_hbm, table_hbm, zero_hbm,
                  id_hbm, acc_out, cnt_out,
                  acc_sh, cnt_sh, src_v, tgt_v, hi_v, lo_v, rows_v, oh_v,
                  sg, so, sa, sc):
        c = lax.axis_index("c")
        s = lax.axis_index("s")
        t0 = s * rpt
        pltpu.sync_copy(zero_hbm, rows_v)
        for k in range(nck):
            pltpu.sync_copy(rows_v.at[pl.ds(0, EB)],
                            acc_sh.at[pl.ds(t0 + k * EB, EB)])

        @pl.when(s < crows_pad // 8)
        def _zero_cnt_sh():
            pltpu.sync_copy(rows_v.at[pl.ds(0, 8), pl.ds(0, EB)],
                            cnt_sh.at[pl.ds(s * 8, 8)])

        plsc.subcore_barrier()

        def chunk_body(g, carry):
            pltpu.sync_copy(src_hbm.at[c, s, pl.ds(g * _CG, _CG)], src_v)
            pltpu.sync_copy(tgt_hbm.at[c, s, pl.ds(g * _CG, _CG)], tgt_v)
            pltpu.sync_copy(hi_hbm.at[c, s, pl.ds(g * _CG, _CG)], hi_v)
            pltpu.sync_copy(lo_hbm.at[c, s, pl.ds(g * _CG, _CG)], lo_v)
            cp_a = cp_c = None
            for j in range(_CG):
                if cp_a is not None:
                    cp_a.wait()
                cp_r = pltpu.async_copy(table_hbm.at[src_v.at[j]], rows_v, sg)
                if cp_c is not None:
                    cp_c.wait()
                cp_o = pltpu.async_copy(id_hbm.at[lo_v.at[j]], oh_v, so)
                cp_r.wait()
                cp_a = pltpu.async_copy(rows_v, acc_sh.at[tgt_v.at[j]], sa,
                                        add=True)
                cp_o.wait()
                cp_c = pltpu.async_copy(oh_v, cnt_sh.at[hi_v.at[j]], sc,
                                        add=True)
            cp_a.wait()
            cp_c.wait()
            return carry

        lax.fori_loop(0, NB // _CG, chunk_body, 0)
        plsc.subcore_barrier()
        for k in range(nck):
            pltpu.sync_copy(acc_sh.at[pl.ds(t0 + k * EB, EB)],
                            rows_v.at[pl.ds(0, EB)])
            pltpu.sync_copy(rows_v.at[pl.ds(0, EB)],
                            acc_out.at[c, pl.ds(t0 + k * EB, EB)])

        @pl.when(s < crows // 8)
        def _drain_cnt():
            pltpu.sync_copy(cnt_sh.at[pl.ds(s * 8, 8)],
                            oh_v.at[pl.ds(0, 8)])
            pltpu.sync_copy(oh_v.at[pl.ds(0, 8)],
                            cnt_out.at[c, pl.ds(s * 8, 8)])

    zero = jnp.zeros((_EB, D), jnp.float32)
    id128 = jnp.eye(_EB, dtype=jnp.float32)
    return sc_kernel(src, tgt, tgt_hi, tgt_lo, table, zero, id128)


def _normalize(acc, cnt, n_pad, D):
    RB = 1024
    assert n_pad % RB == 0

    def body(a_ref, c_ref, o_ref):
        p = a_ref[0] + a_ref[1]
        n = c_ref[0] + c_ref[1]
        div = jnp.where(n == 0.0, 1.0, n)
        o_ref[...] = p / div + 1e-8

    return pl.pallas_call(
        body,
        grid=(n_pad // RB,),
        in_specs=[
            pl.BlockSpec((2, RB, D), lambda r: (0, r, 0)),
            pl.BlockSpec((2, RB, 1), lambda r: (0, r, 0)),
        ],
        out_specs=pl.BlockSpec((RB, D), lambda r: (r, 0)),
        out_shape=jax.ShapeDtypeStruct((n_pad, D), jnp.float32),
    )(acc, cnt)


def kernel(edge_lists, node_states, W, b):
    E, M, _ = edge_lists.shape
    N, D = node_states.shape

    table = _transform_table(node_states, W, b, E)

    el = edge_lists.astype(jnp.int32)
    src = (el[:, :, 0] + (jnp.arange(E, dtype=jnp.int32) * N)[:, None]).reshape(-1)
    tgt = el[:, :, 1].reshape(-1)

    lane_total = _NC * _NS * _EB * _CG
    m_tot = E * M
    m_pad = -(-m_tot // lane_total) * lane_total
    n_pad = -(-(N + 1) // (_NS * _EB)) * (_NS * _EB)
    src = jnp.pad(src, (0, m_pad - m_tot))
    tgt = jnp.pad(tgt, (0, m_pad - m_tot), constant_values=N)
    NB = m_pad // (_NC * _NS * _EB)
    shape4 = (_NC, _NS, NB, _EB)
    src = src.reshape(shape4)
    tgt = tgt.reshape(shape4)
    tgt_hi = jax.lax.shift_right_logical(tgt, 7)
    tgt_lo = jax.lax.bitwise_and(tgt, 127)

    acc, cnt = _scatter_accumulate(src, tgt, tgt_hi, tgt_lo, table, n_pad, D)
    out = _normalize(acc, cnt.reshape(_NC, n_pad, 1), n_pad, D)
    return out[:N]


# speedup vs baseline: 2.1102x; 1.3546x over previous
import functools

import jax
import jax.numpy as jnp
from jax import lax
from jax.experimental import pallas as pl
from jax.experimental.pallas import tpu as pltpu
from jax.experimental.pallas import tpu_sc as plsc

_NC = 2
_NS = 16
_EB = 64
_CG = 8
_RW = 128


def _transform_table(node_states, W, b, E):
    N, D = node_states.shape
    RB = 2000
    assert N % RB == 0

    def body(x_ref, w_ref, b_ref, o_ref):
        acc = lax.dot_general(
            x_ref[...], w_ref[...], (((1,), (1,)), ((), ())),
            preferred_element_type=jnp.float32)
        o_ref[...] = (acc + b_ref[0])[None]

    out = pl.pallas_call(
        body,
        grid=(E, N // RB),
        in_specs=[
            pl.BlockSpec((RB, D), lambda e, r: (r, 0)),
            pl.BlockSpec((D, D), lambda e, r: (e, 0)),
            pl.BlockSpec((1, 1, D), lambda e, r: (e, 0, 0)),
        ],
        out_specs=pl.BlockSpec((1, RB, D), lambda e, r: (e, r, 0)),
        out_shape=jax.ShapeDtypeStruct((E, N, D), jnp.float32),
    )(node_states, W, b.reshape(E, 1, D))
    return out.reshape(E * N, D)


def _scatter_accumulate(src, tgt, tgt_hi, tgt_lo, table, n_pad, D):
    NC, NS, NB, EB = src.shape
    rpt = n_pad // NS
    nck = rpt // EB
    crows = n_pad // _RW
    crows_pad = -(-crows // 8) * 8
    mesh = plsc.VectorSubcoreMesh(core_axis_name="c", subcore_axis_name="s")

    @functools.partial(
        pl.kernel,
        mesh=mesh,
        out_type=[
            jax.ShapeDtypeStruct((NC, n_pad, D), jnp.float32),
            jax.ShapeDtypeStruct((NC, crows, _RW), jnp.float32),
        ],
        scratch_types=[
            pltpu.VMEM_SHARED((n_pad, D), jnp.float32),
            pltpu.VMEM_SHARED((crows_pad, _RW), jnp.float32),
            pltpu.VMEM((_CG, EB), jnp.int32),
            pltpu.VMEM((_CG, EB), jnp.int32),
            pltpu.VMEM((_CG, EB), jnp.int32),
            pltpu.VMEM((_CG, EB), jnp.int32),
            pltpu.VMEM((EB, D), jnp.float32),
            pltpu.VMEM((EB, D), jnp.float32),
            pltpu.VMEM((EB, _RW), jnp.float32),
            pltpu.VMEM((EB, _RW), jnp.float32),
            [pltpu.SemaphoreType.DMA] * 8,
        ],
    )
    def sc_kernel(src_hbm, tgt_hbm, hi_hbm, lo_hbm, table_hbm, zero_hbm,
                  id_hbm, acc_out, cnt_out,
                  acc_sh, cnt_sh, src_v, tgt_v, hi_v, lo_v,
                  rows0, rows1, oh0, oh1, sems):
        c = lax.axis_index("c")
        s = lax.axis_index("s")
        t0 = s * rpt
        rows = (rows0, rows1)
        oh = (oh0, oh1)
        sgm, som, sam, scm = sems[0:2], sems[2:4], sems[4:6], sems[6:8]
        pltpu.sync_copy(zero_hbm, rows0)
        for k in range(nck):
            pltpu.sync_copy(rows0.at[pl.ds(0, EB)],
                            acc_sh.at[pl.ds(t0 + k * EB, EB)])

        @pl.when(s < crows_pad // 8)
        def _zero_cnt_sh():
            pltpu.sync_copy(rows0.at[pl.ds(0, 8)],
                            cnt_sh.at[pl.ds(s * 8, 8)])

        plsc.subcore_barrier()

        def chunk_body(g, carry):
            pltpu.sync_copy(src_hbm.at[c, s, pl.ds(g * _CG, _CG)], src_v)
            pltpu.sync_copy(tgt_hbm.at[c, s, pl.ds(g * _CG, _CG)], tgt_v)
            pltpu.sync_copy(hi_hbm.at[c, s, pl.ds(g * _CG, _CG)], hi_v)
            pltpu.sync_copy(lo_hbm.at[c, s, pl.ds(g * _CG, _CG)], lo_v)
            sca = [None, None]
            scc = [None, None]
            for j in range(_CG):
                b = j & 1
                if sca[b] is not None:
                    sca[b].wait()
                cp_r = pltpu.async_copy(table_hbm.at[src_v.at[j]], rows[b],
                                        sgm[b])
                if scc[b] is not None:
                    scc[b].wait()
                cp_o = pltpu.async_copy(id_hbm.at[lo_v.at[j]], oh[b], som[b])
                cp_r.wait()
                sca[b] = pltpu.async_copy(rows[b], acc_sh.at[tgt_v.at[j]],
                                          sam[b], add=True)
                cp_o.wait()
                scc[b] = pltpu.async_copy(oh[b], cnt_sh.at[hi_v.at[j]],
                                          scm[b], add=True)
            for b in range(2):
                sca[b].wait()
                scc[b].wait()
            return carry

        lax.fori_loop(0, NB // _CG, chunk_body, 0)
        plsc.subcore_barrier()
        for k in range(nck):
            pltpu.sync_copy(acc_sh.at[pl.ds(t0 + k * EB, EB)],
                            rows0.at[pl.ds(0, EB)])
            pltpu.sync_copy(rows0.at[pl.ds(0, EB)],
                            acc_out.at[c, pl.ds(t0 + k * EB, EB)])

        @pl.when(s < crows // 8)
        def _drain_cnt():
            pltpu.sync_copy(cnt_sh.at[pl.ds(s * 8, 8)],
                            oh0.at[pl.ds(0, 8)])
            pltpu.sync_copy(oh0.at[pl.ds(0, 8)],
                            cnt_out.at[c, pl.ds(s * 8, 8)])

    zero = jnp.zeros((_EB, D), jnp.float32)
    id128 = jnp.eye(_RW, dtype=jnp.float32)
    return sc_kernel(src, tgt, tgt_hi, tgt_lo, table, zero, id128)


def _normalize(acc, cnt, n_pad, D):
    RB = 1024
    assert n_pad % RB == 0

    def body(a_ref, c_ref, o_ref):
        p = a_ref[0] + a_ref[1]
        n = c_ref[0] + c_ref[1]
        div = jnp.where(n == 0.0, 1.0, n)
        o_ref[...] = p / div + 1e-8

    return pl.pallas_call(
        body,
        grid=(n_pad // RB,),
        in_specs=[
            pl.BlockSpec((2, RB, D), lambda r: (0, r, 0)),
            pl.BlockSpec((2, RB, 1), lambda r: (0, r, 0)),
        ],
        out_specs=pl.BlockSpec((RB, D), lambda r: (r, 0)),
        out_shape=jax.ShapeDtypeStruct((n_pad, D), jnp.float32),
    )(acc, cnt)


def kernel(edge_lists, node_states, W, b):
    E, M, _ = edge_lists.shape
    N, D = node_states.shape

    table = _transform_table(node_states, W, b, E)

    el = edge_lists.astype(jnp.int32)
    src = (el[:, :, 0] + (jnp.arange(E, dtype=jnp.int32) * N)[:, None]).reshape(-1)
    tgt = el[:, :, 1].reshape(-1)

    lane_total = _NC * _NS * _EB * _CG
    m_tot = E * M
    m_pad = -(-m_tot // lane_total) * lane_total
    n_pad = -(-(N + 1) // (_NS * _EB)) * (_NS * _EB)
    src = jnp.pad(src, (0, m_pad - m_tot))
    tgt = jnp.pad(tgt, (0, m_pad - m_tot), constant_values=N)
    NB = m_pad // (_NC * _NS * _EB)
    shape4 = (_NC, _NS, NB, _EB)
    src = src.reshape(shape4)
    tgt = tgt.reshape(shape4)
    tgt_hi = jax.lax.shift_right_logical(tgt, 7)
    tgt_lo = jax.lax.bitwise_and(tgt, 127)

    acc, cnt = _scatter_accumulate(src, tgt, tgt_hi, tgt_lo, table, n_pad, D)
    out = _normalize(acc, cnt.reshape(_NC, n_pad, 1), n_pad, D)
    return out[:N]
